# baseline (speedup 1.0000x reference)
import functools

import jax
import jax.numpy as jnp
from jax import lax
from jax.experimental import pallas as pl
from jax.experimental.pallas import tpu as pltpu
from jax.experimental.pallas import tpu_sc as plsc

_D = 64
_B = 16384
_NC = 2
_NS = 16
_L = 16
_NW = _NC * _NS
_BPW = _B // _NW


def _sc_glove(left_hbm, right_hbm, lemb_hbm, lbias_hbm, remb_hbm, rbias_hbm,
              out_hbm, idx_l, idx_r, lrows, rrows, lb, rb, tbuf, outv, sem):
    wid = lax.axis_index("s") * _NC + lax.axis_index("c")
    base = wid * _BPW

    pltpu.sync_copy(left_hbm.at[pl.ds(base, _BPW)], idx_l)
    pltpu.sync_copy(right_hbm.at[pl.ds(base, _BPW)], idx_r)

    c1 = pltpu.async_copy(lemb_hbm.at[idx_l], lrows, sem)
    c2 = pltpu.async_copy(remb_hbm.at[idx_r], rrows, sem)
    c3 = pltpu.async_copy(lbias_hbm.at[idx_l], lb, sem)
    c4 = pltpu.async_copy(rbias_hbm.at[idx_r], rb, sem)
    c1.wait()
    c2.wait()
    c3.wait()
    c4.wait()

    iota = lax.iota(jnp.int32, _L)
    col0 = iota * _L

    def group(g, carry):
        eb = g * _L
        for j in range(_L):
            e = eb + j
            p = lrows[e, pl.ds(0, _L)] * rrows[e, pl.ds(0, _L)]
            for k in range(1, _D // _L):
                p = p + lrows[e, pl.ds(k * _L, _L)] * rrows[e, pl.ds(k * _L, _L)]
            tbuf[pl.ds(j * _L, _L)] = p
        acc = plsc.load_gather(tbuf, [col0])
        for t in range(1, _L):
            acc = acc + plsc.load_gather(tbuf, [col0 + t])
        outv[pl.ds(eb, _L)] = acc + lb[pl.ds(eb, _L)] + rb[pl.ds(eb, _L)]
        return carry

    lax.fori_loop(0, _BPW // _L, group, 0)
    pltpu.sync_copy(outv, out_hbm.at[pl.ds(base, _BPW)])


@functools.cache
def _build():
    mesh = plsc.VectorSubcoreMesh(core_axis_name="c", subcore_axis_name="s")
    return pl.kernel(
        _sc_glove,
        mesh=mesh,
        compiler_params=pltpu.CompilerParams(
            needs_layout_passes=False, use_tc_tiling_on_sc=False),
        out_type=jax.ShapeDtypeStruct((_B,), jnp.float32),
        scratch_types=[
            pltpu.VMEM((_BPW,), jnp.int32),
            pltpu.VMEM((_BPW,), jnp.int32),
            pltpu.VMEM((_BPW, _D), jnp.float32),
            pltpu.VMEM((_BPW, _D), jnp.float32),
            pltpu.VMEM((_BPW,), jnp.float32),
            pltpu.VMEM((_BPW,), jnp.float32),
            pltpu.VMEM((_L * _L,), jnp.float32),
            pltpu.VMEM((_BPW,), jnp.float32),
            pltpu.SemaphoreType.DMA,
        ],
    )


def kernel(left, right, l_emb, l_bias, r_emb, r_bias):
    return _build()(left.astype(jnp.int32), right.astype(jnp.int32),
                    l_emb, l_bias.reshape(-1), r_emb, r_bias.reshape(-1))
